# baseline (speedup 1.0000x reference)
import functools

import jax
import jax.numpy as jnp
from jax.experimental import pallas as pl
from jax.experimental.pallas import tpu as pltpu
from jax.experimental.pallas import tpu_sc as plsc

C = 1024
K = 9
D = C * K
R = 512
STEPS = D // R
SIZE = 512

NSUB = 16
CPW = C // NSUB
SUNROLL = 4

_HI = jax.lax.Precision.HIGHEST


def _reduce_body(v1_ref, v2_ref, imp_ref, acc_ref, cs2_ref):
    i = pl.program_id(0)
    half = (i % 2) * R

    rows = jnp.sum(jnp.abs(v1_ref[...]), axis=1, keepdims=True)

    @pl.when(i < 2)
    def _():
        acc_ref[pl.ds(half, R), :] = rows

    @pl.when(i >= 2)
    def _():
        acc_ref[pl.ds(half, R), :] = acc_ref[pl.ds(half, R), :] + rows

    colpart = jnp.sum(jnp.abs(v2_ref[...]), axis=0, keepdims=True)

    @pl.when(i == 0)
    def _():
        cs2_ref[...] = colpart

    @pl.when(i > 0)
    def _():
        cs2_ref[...] = cs2_ref[...] + colpart

    @pl.when(i == STEPS - 1)
    def _():
        eye = (jax.lax.broadcasted_iota(jnp.int32, (C, C), 0)
               == jax.lax.broadcasted_iota(jnp.int32, (C, C), 1)
               ).astype(jnp.float32)
        imp1_row = jax.lax.dot_general(
            acc_ref[...], eye, (((0,), (0,)), ((), ())), precision=_HI)
        imp_ref[...] = imp1_row + cs2_ref[...]


def _sc_select_body(imp_hbm, out_hbm, imp_v, cnt_st, counts_v, posv, vals,
                    outv, outstage_sh, counts_sh, sem):
    wid = jax.lax.axis_index("s")
    base = wid * CPW

    pltpu.sync_copy(imp_hbm.at[0], imp_v)

    tgts = tuple(imp_v[pl.ds(base + 16 * j, 16)] for j in range(4))
    cidx = tuple(jax.lax.iota(jnp.int32, 16) + (base + 16 * j)
                 for j in range(4))
    ranks = tuple(jnp.zeros((16,), jnp.int32) for _ in range(4))

    def sbody(g, ranks):
        srcv = imp_v[pl.ds(16 * g, 16)]
        for u in range(16):
            sv = srcv[u]
            s = 16 * g + u
            ranks = tuple(
                r + jnp.where((sv < t) | ((sv == t) & (s < ci)), 1, 0)
                for r, t, ci in zip(ranks, tgts, cidx))
        return ranks

    ranks = jax.lax.fori_loop(0, C // 16, sbody, ranks)

    masks = tuple(r < SIZE for r in ranks)
    mints = tuple(jnp.where(m, 1.0, 0.0) for m in masks)
    lane = jax.lax.iota(jnp.int32, 16)
    excls = []
    subcnts = []
    for m in mints:
        run = jnp.float32(0)
        excl = jnp.zeros((16,), jnp.float32)
        for u in range(16):
            excl = jnp.where(lane == u, run, excl)
            run = run + m[u]
        excls.append(excl)
        subcnts.append(run)
    cnt = subcnts[0] + subcnts[1] + subcnts[2] + subcnts[3]

    cnt_st[...] = jnp.zeros((16,), jnp.float32) + cnt
    pltpu.sync_copy(cnt_st, counts_sh.at[pl.ds(wid * 16, 16)])
    plsc.subcore_barrier()
    pltpu.sync_copy(counts_sh, counts_v)

    running = jnp.float32(0)
    for k in range(NSUB):
        ck = counts_v[pl.ds(16 * k, 16)][0]
        running = running + jnp.where(wid > k, ck, 0.0)

    dump = jnp.float32(SIZE) + wid.astype(jnp.float32)
    for j in range(4):
        excl = excls[j]
        pos = jnp.where(masks[j], excl + running, dump)
        running = running + subcnts[j]
        lin = (cidx[j].astype(jnp.float32) * (2.0 / float(C - 1)) - 1.0)
        posv[pl.ds(16 * j, 16)] = pos.astype(jnp.int32)
        vals[pl.ds(16 * j, 16)] = lin

    pltpu.sync_copy(vals, outstage_sh.at[posv])
    plsc.subcore_barrier()

    @pl.when(wid == 0)
    def _():
        pltpu.sync_copy(outstage_sh.at[pl.ds(0, SIZE)], outv)
        pltpu.sync_copy(outv, out_hbm)


_sc_select = functools.partial(
    pl.kernel,
    out_type=jax.ShapeDtypeStruct((SIZE,), jnp.float32),
    mesh=plsc.VectorSubcoreMesh(core_axis_name="c", subcore_axis_name="s",
                                num_cores=1),
    scratch_types=[
        pltpu.VMEM((C,), jnp.float32),
        pltpu.VMEM((16,), jnp.float32),
        pltpu.VMEM((NSUB * 16,), jnp.float32),
        pltpu.VMEM((CPW,), jnp.int32),
        pltpu.VMEM((CPW,), jnp.float32),
        pltpu.VMEM((SIZE,), jnp.float32),
        pltpu.VMEM_SHARED((SIZE + NSUB,), jnp.float32),
        pltpu.VMEM_SHARED((NSUB * 16,), jnp.float32),
        pltpu.SemaphoreType.DMA,
    ],
)(_sc_select_body)


@jax.jit
def _run(w1, w2):
    v1 = jnp.transpose(w1, (2, 3, 0, 1)).reshape(D, C)
    v2 = jnp.transpose(w2, (2, 3, 0, 1)).reshape(D, C)

    imp = pl.pallas_call(
        _reduce_body,
        grid=(STEPS,),
        in_specs=[
            pl.BlockSpec((R, C), lambda i: (i, 0)),
            pl.BlockSpec((R, C), lambda i: (i, 0)),
        ],
        out_specs=pl.BlockSpec((1, C), lambda i: (0, 0)),
        out_shape=jax.ShapeDtypeStruct((1, C), jnp.float32),
        scratch_shapes=[
            pltpu.VMEM((C, 1), jnp.float32),
            pltpu.VMEM((1, C), jnp.float32),
        ],
        compiler_params=pltpu.CompilerParams(
            dimension_semantics=("arbitrary",),
        ),
    )(v1, v2)

    return _sc_select(imp)


def kernel(w1, w2, size):
    return _run(w1, w2) + size * 0


# speedup vs baseline: 2.2289x; 2.2289x over previous
import functools

import jax
import jax.numpy as jnp
from jax.experimental import pallas as pl
from jax.experimental.pallas import tpu as pltpu
from jax.experimental.pallas import tpu_sc as plsc

C = 1024
K = 9
D = C * K
R = 512
STEPS = D // R
SIZE = 512

NSUB = 16
CPW = C // NSUB
SUNROLL = 4

_HI = jax.lax.Precision.HIGHEST


def _reduce_body(v1_ref, v2_ref, imp_ref, acc_ref, cs2_ref):
    i = pl.program_id(0)
    half = (i % 2) * R

    rows = jnp.sum(jnp.abs(v1_ref[...]), axis=1, keepdims=True)

    @pl.when(i < 2)
    def _():
        acc_ref[pl.ds(half, R), :] = rows

    @pl.when(i >= 2)
    def _():
        acc_ref[pl.ds(half, R), :] = acc_ref[pl.ds(half, R), :] + rows

    colpart = jnp.sum(jnp.abs(v2_ref[...]), axis=0, keepdims=True)

    @pl.when(i == 0)
    def _():
        cs2_ref[...] = colpart

    @pl.when(i > 0)
    def _():
        cs2_ref[...] = cs2_ref[...] + colpart

    @pl.when(i == STEPS - 1)
    def _():
        eye = (jax.lax.broadcasted_iota(jnp.int32, (C, C), 0)
               == jax.lax.broadcasted_iota(jnp.int32, (C, C), 1)
               ).astype(jnp.float32)
        imp1_row = jax.lax.dot_general(
            acc_ref[...], eye, (((0,), (0,)), ((), ())), precision=_HI)
        imp_ref[...] = imp1_row + cs2_ref[...]


def _sc_select_body(imp_hbm, out_hbm, imp_v, cnt_st, counts_v, posv, vals,
                    outv, outstage_sh, counts_sh, sem):
    wid = jax.lax.axis_index("s")
    base = wid * CPW

    pltpu.sync_copy(imp_hbm.at[0], imp_v.at[pl.ds(0, C)])
    pltpu.sync_copy(imp_hbm.at[0], imp_v.at[pl.ds(C, C)])

    tgts = tuple(imp_v[pl.ds(base + 16 * j, 16)] for j in range(4))
    cidx = tuple(jax.lax.iota(jnp.int32, 16) + (base + 16 * j)
                 for j in range(4))
    ranks = tuple(jnp.zeros((16,), jnp.int32) for _ in range(4))

    def obody(o, ranks):
        ranks = tuple(
            r + jnp.where(
                (imp_v[pl.ds(base + 16 * j + C - o, 16)] < t)
                | ((imp_v[pl.ds(base + 16 * j + C - o, 16)] == t)
                   & (ci >= o)),
                1, 0)
            for j, (r, t, ci) in enumerate(zip(ranks, tgts, cidx)))
        return ranks

    ranks = jax.lax.fori_loop(1, C, obody, ranks)

    masks = tuple(r < SIZE for r in ranks)
    mints = tuple(jnp.where(m, 1.0, 0.0) for m in masks)
    lane = jax.lax.iota(jnp.int32, 16)
    excls = []
    subcnts = []
    for m in mints:
        run = jnp.float32(0)
        excl = jnp.zeros((16,), jnp.float32)
        for u in range(16):
            excl = jnp.where(lane == u, run, excl)
            run = run + m[u]
        excls.append(excl)
        subcnts.append(run)
    cnt = subcnts[0] + subcnts[1] + subcnts[2] + subcnts[3]

    cnt_st[...] = jnp.zeros((16,), jnp.float32) + cnt
    pltpu.sync_copy(cnt_st, counts_sh.at[pl.ds(wid * 16, 16)])
    plsc.subcore_barrier()
    pltpu.sync_copy(counts_sh, counts_v)

    running = jnp.float32(0)
    for k in range(NSUB):
        ck = counts_v[pl.ds(16 * k, 16)][0]
        running = running + jnp.where(wid > k, ck, 0.0)

    dump = jnp.float32(SIZE) + wid.astype(jnp.float32)
    for j in range(4):
        excl = excls[j]
        pos = jnp.where(masks[j], excl + running, dump)
        running = running + subcnts[j]
        lin = (cidx[j].astype(jnp.float32) * (2.0 / float(C - 1)) - 1.0)
        posv[pl.ds(16 * j, 16)] = pos.astype(jnp.int32)
        vals[pl.ds(16 * j, 16)] = lin

    pltpu.sync_copy(vals, outstage_sh.at[posv])
    plsc.subcore_barrier()

    @pl.when(wid == 0)
    def _():
        pltpu.sync_copy(outstage_sh.at[pl.ds(0, SIZE)], outv)
        pltpu.sync_copy(outv, out_hbm)


_sc_select = functools.partial(
    pl.kernel,
    out_type=jax.ShapeDtypeStruct((SIZE,), jnp.float32),
    mesh=plsc.VectorSubcoreMesh(core_axis_name="c", subcore_axis_name="s",
                                num_cores=1),
    scratch_types=[
        pltpu.VMEM((2 * C,), jnp.float32),
        pltpu.VMEM((16,), jnp.float32),
        pltpu.VMEM((NSUB * 16,), jnp.float32),
        pltpu.VMEM((CPW,), jnp.int32),
        pltpu.VMEM((CPW,), jnp.float32),
        pltpu.VMEM((SIZE,), jnp.float32),
        pltpu.VMEM_SHARED((SIZE + NSUB,), jnp.float32),
        pltpu.VMEM_SHARED((NSUB * 16,), jnp.float32),
        pltpu.SemaphoreType.DMA,
    ],
)(_sc_select_body)


@jax.jit
def _run(w1, w2):
    v1 = jnp.transpose(w1, (2, 3, 0, 1)).reshape(D, C)
    v2 = jnp.transpose(w2, (2, 3, 0, 1)).reshape(D, C)

    imp = pl.pallas_call(
        _reduce_body,
        grid=(STEPS,),
        in_specs=[
            pl.BlockSpec((R, C), lambda i: (i, 0)),
            pl.BlockSpec((R, C), lambda i: (i, 0)),
        ],
        out_specs=pl.BlockSpec((1, C), lambda i: (0, 0)),
        out_shape=jax.ShapeDtypeStruct((1, C), jnp.float32),
        scratch_shapes=[
            pltpu.VMEM((C, 1), jnp.float32),
            pltpu.VMEM((1, C), jnp.float32),
        ],
        compiler_params=pltpu.CompilerParams(
            dimension_semantics=("arbitrary",),
        ),
    )(v1, v2)

    return _sc_select(imp)


def kernel(w1, w2, size):
    return _run(w1, w2) + size * 0


# speedup vs baseline: 4.0517x; 1.8178x over previous
import jax
import jax.numpy as jnp
from jax.experimental import pallas as pl
from jax.experimental.pallas import tpu as pltpu

C = 1024
K = 9
D = C * K
R = 512
STEPS = D // R
SIZE = 512

_HI = jax.lax.Precision.HIGHEST


def _body(v1_ref, v2_ref, out_ref, acc_ref, cs2_ref):
    i = pl.program_id(0)
    half = (i % 2) * R

    rows = jnp.sum(jnp.abs(v1_ref[...]), axis=1, keepdims=True)

    @pl.when(i < 2)
    def _():
        acc_ref[pl.ds(half, R), :] = rows

    @pl.when(i >= 2)
    def _():
        acc_ref[pl.ds(half, R), :] = acc_ref[pl.ds(half, R), :] + rows

    colpart = jnp.sum(jnp.abs(v2_ref[...]), axis=0, keepdims=True)

    @pl.when(i == 0)
    def _():
        cs2_ref[...] = colpart

    @pl.when(i > 0)
    def _():
        cs2_ref[...] = cs2_ref[...] + colpart

    @pl.when(i == STEPS - 1)
    def _():
        imp1_col = acc_ref[...]
        imp2_row = cs2_ref[...]
        eye = (jax.lax.broadcasted_iota(jnp.int32, (C, C), 0)
               == jax.lax.broadcasted_iota(jnp.int32, (C, C), 1)
               ).astype(jnp.float32)
        imp1_row = jax.lax.dot_general(
            imp1_col, eye, (((0,), (0,)), ((), ())), precision=_HI)
        imp2_col = jax.lax.dot_general(
            eye, imp2_row, (((1,), (1,)), ((), ())), precision=_HI)
        imp_col = imp1_col + imp2_col
        imp_row = imp1_row + imp2_row

        src_i = jax.lax.broadcasted_iota(jnp.int32, (C, C), 1)
        tgt_i = jax.lax.broadcasted_iota(jnp.int32, (C, C), 0)
        sel = (imp_row < imp_col) | ((imp_row == imp_col) & (src_i < tgt_i))
        rank = jnp.sum(jnp.where(sel, 1.0, 0.0), axis=1, keepdims=True)
        maskf = jnp.where(rank < float(SIZE), 1.0, 0.0)

        lower = jnp.where(src_i < tgt_i, 1.0, 0.0).astype(jnp.bfloat16)
        pos = jax.lax.dot_general(
            lower, maskf.astype(jnp.bfloat16), (((1,), (0,)), ((), ())),
            preferred_element_type=jnp.float32)

        slot = jax.lax.broadcasted_iota(
            jnp.int32, (C, SIZE), 1).astype(jnp.float32)
        w = maskf * jnp.where(pos == slot, 1.0, 0.0)
        lin = (-1.0 + jax.lax.broadcasted_iota(jnp.int32, (C, 1), 0)
               .astype(jnp.float32) * (2.0 / float(C - 1)))
        out_ref[...] = jnp.sum(w * lin, axis=0, keepdims=True)


@jax.jit
def _run(w1, w2):
    v1 = jnp.transpose(w1, (2, 3, 0, 1)).reshape(D, C)
    v2 = jnp.transpose(w2, (2, 3, 0, 1)).reshape(D, C)

    return pl.pallas_call(
        _body,
        grid=(STEPS,),
        in_specs=[
            pl.BlockSpec((R, C), lambda i: (i, 0)),
            pl.BlockSpec((R, C), lambda i: (i, 0)),
        ],
        out_specs=pl.BlockSpec((1, SIZE), lambda i: (0, 0)),
        out_shape=jax.ShapeDtypeStruct((1, SIZE), jnp.float32),
        scratch_shapes=[
            pltpu.VMEM((C, 1), jnp.float32),
            pltpu.VMEM((1, C), jnp.float32),
        ],
        compiler_params=pltpu.CompilerParams(
            dimension_semantics=("arbitrary",),
        ),
    )(v1, v2)


def kernel(w1, w2, size):
    return _run(w1, w2).reshape(SIZE) + size * 0


# speedup vs baseline: 4.2743x; 1.0550x over previous
import jax
import jax.numpy as jnp
from jax.experimental import pallas as pl
from jax.experimental.pallas import tpu as pltpu

C = 1024
K = 9
D = C * K
R = 1024
STEPS = D // R
SIZE = 512

_HI = jax.lax.Precision.HIGHEST


def _body(v1_ref, v2_ref, out_ref, acc_ref, cs2_ref):
    i = pl.program_id(0)

    rows = jnp.sum(jnp.abs(v1_ref[...]), axis=1, keepdims=True)

    @pl.when(i == 0)
    def _():
        acc_ref[...] = rows

    @pl.when(i > 0)
    def _():
        acc_ref[...] = acc_ref[...] + rows

    colpart = jnp.sum(jnp.abs(v2_ref[...]), axis=0, keepdims=True)

    @pl.when(i == 0)
    def _():
        cs2_ref[...] = colpart

    @pl.when(i > 0)
    def _():
        cs2_ref[...] = cs2_ref[...] + colpart

    @pl.when(i == STEPS - 1)
    def _():
        imp1_col = acc_ref[...]
        imp2_row = cs2_ref[...]
        eye = (jax.lax.broadcasted_iota(jnp.int32, (C, C), 0)
               == jax.lax.broadcasted_iota(jnp.int32, (C, C), 1)
               ).astype(jnp.float32)
        imp1_row = jax.lax.dot_general(
            imp1_col, eye, (((0,), (0,)), ((), ())), precision=_HI)
        imp2_col = jax.lax.dot_general(
            eye, imp2_row, (((1,), (1,)), ((), ())), precision=_HI)
        imp_col = imp1_col + imp2_col
        imp_row = imp1_row + imp2_row

        src_i = jax.lax.broadcasted_iota(jnp.int32, (C, C), 1)
        tgt_i = jax.lax.broadcasted_iota(jnp.int32, (C, C), 0)
        sel = (imp_row < imp_col) | ((imp_row == imp_col) & (src_i < tgt_i))
        rank = jnp.sum(jnp.where(sel, 1.0, 0.0), axis=1, keepdims=True)
        maskf = jnp.where(rank < float(SIZE), 1.0, 0.0)

        lower = jnp.where(src_i < tgt_i, 1.0, 0.0).astype(jnp.bfloat16)
        pos = jax.lax.dot_general(
            lower, maskf.astype(jnp.bfloat16), (((1,), (0,)), ((), ())),
            preferred_element_type=jnp.float32)

        slot = jax.lax.broadcasted_iota(
            jnp.int32, (C, SIZE), 1).astype(jnp.float32)
        w = maskf * jnp.where(pos == slot, 1.0, 0.0)
        lin = (-1.0 + jax.lax.broadcasted_iota(jnp.int32, (C, 1), 0)
               .astype(jnp.float32) * (2.0 / float(C - 1)))
        out_ref[...] = jnp.sum(w * lin, axis=0, keepdims=True)


@jax.jit
def _run(w1, w2):
    v1 = jnp.transpose(w1, (2, 3, 0, 1)).reshape(D, C)
    v2 = jnp.transpose(w2, (2, 3, 0, 1)).reshape(D, C)

    return pl.pallas_call(
        _body,
        grid=(STEPS,),
        in_specs=[
            pl.BlockSpec((R, C), lambda i: (i, 0)),
            pl.BlockSpec((R, C), lambda i: (i, 0)),
        ],
        out_specs=pl.BlockSpec((1, SIZE), lambda i: (0, 0)),
        out_shape=jax.ShapeDtypeStruct((1, SIZE), jnp.float32),
        scratch_shapes=[
            pltpu.VMEM((C, 1), jnp.float32),
            pltpu.VMEM((1, C), jnp.float32),
        ],
        compiler_params=pltpu.CompilerParams(
            dimension_semantics=("arbitrary",),
        ),
    )(v1, v2)


def kernel(w1, w2, size):
    return _run(w1, w2).reshape(SIZE) + size * 0
